# baseline (speedup 1.0000x reference)
import functools

import jax
import jax.numpy as jnp
from jax import lax
from jax.experimental import pallas as pl
from jax.experimental.pallas import tpu as pltpu
from jax.experimental.pallas import tpu_sc as plsc

NFIELD = 26
VOCAB = 100000
VOCABP = 100096
EMB = 16
BATCH = 4096
NCORE = 2
NSUB = 16
NWORK = NCORE * NSUB
BPW = BATCH // NWORK
NLOOK = NFIELD * BPW
EROWS = NFIELD * VOCAB * EMB // 128
WROWS = NFIELD * VOCABP // 128


def _fm_body(eridx_hbm, esoff_hbm, wridx_hbm, wsoff_hbm, emb_hbm, w_hbm,
             dproj_hbm, dlin_hbm, out_hbm,
             eridx_v, esoff_v, wridx_v, wsoff_v, ebuf_v, wbuf_v,
             dproj_v, dlin_v, accs_v, accq_v, out_v, esem, wsem):
    wid = lax.axis_index("s") * NCORE + lax.axis_index("c")
    base = wid * BPW

    pltpu.sync_copy(eridx_hbm.at[wid], eridx_v)
    pltpu.sync_copy(esoff_hbm.at[wid], esoff_v)
    pltpu.sync_copy(wridx_hbm.at[wid], wridx_v)
    pltpu.sync_copy(wsoff_hbm.at[wid], wsoff_v)
    pltpu.sync_copy(dproj_hbm.at[wid], dproj_v)
    pltpu.sync_copy(dlin_hbm.at[pl.ds(base, BPW)], dlin_v)

    def init_body(b, carry):
        o = pl.multiple_of(b * 16, 16)
        row = dproj_v[pl.ds(o, 16)]
        accs_v[pl.ds(o, 16)] = row
        accq_v[pl.ds(o, 16)] = row * row
        return carry

    lax.fori_loop(0, BPW, init_body, 0)

    def fire(f, p):
        o = pl.multiple_of(f * BPW, BPW)
        pltpu.async_copy(
            emb_hbm.at[eridx_v.at[pl.ds(o, BPW)]], ebuf_v.at[p], esem)
        pltpu.async_copy(
            w_hbm.at[wridx_v.at[pl.ds(o, BPW)]], wbuf_v.at[p], wsem)

    fire(0, 0)
    fire(1, 1)

    lane = lax.iota(jnp.int32, 16)

    def field_body(f, carry):
        p = lax.bitwise_and(f, 1)
        pltpu.make_async_copy(
            emb_hbm.at[pl.ds(0, BPW)], ebuf_v.at[p], esem).wait()
        pltpu.make_async_copy(
            w_hbm.at[pl.ds(0, BPW)], wbuf_v.at[p], wsem).wait()

        fbase = pl.multiple_of(f * BPW, BPW)

        def row_body(g, cc):
            g16 = pl.multiple_of(g * 16, 16)
            off16 = esoff_v[pl.ds(fbase + g16, 16)]
            for i in range(16):
                off = pl.multiple_of(off16[i], 16)
                r = ebuf_v[p, g16 + i, pl.ds(off, 16)]
                o = pl.multiple_of((g16 + i) * 16, 16)
                accs_v[pl.ds(o, 16)] = accs_v[pl.ds(o, 16)] + r
                accq_v[pl.ds(o, 16)] = accq_v[pl.ds(o, 16)] + r * r
            return cc

        lax.fori_loop(0, BPW // 16, row_body, 0)

        pvec = jnp.full((16,), p, jnp.int32)
        for g in range(BPW // 16):
            woff = wsoff_v[pl.ds(fbase + g * 16, 16)]
            wv = plsc.load_gather(wbuf_v, [pvec, g * 16 + lane, woff])
            dlin_v[pl.ds(g * 16, 16)] = dlin_v[pl.ds(g * 16, 16)] + wv

        @pl.when(f + 2 < NFIELD)
        def _():
            fire(f + 2, p)

        return carry

    lax.fori_loop(0, NFIELD, field_body, 0)

    for g in range(BPW // 16):
        fm = jnp.zeros((16,), jnp.float32)
        for i in range(16):
            o = (g * 16 + i) * 16
            sv = accs_v[pl.ds(o, 16)]
            qv = accq_v[pl.ds(o, 16)]
            fm = jnp.where(lane == i, jnp.sum(sv * sv - qv), fm)
        logit = dlin_v[pl.ds(g * 16, 16)] + 0.5 * fm
        out_v[pl.ds(g * 16, 16)] = 1.0 / (1.0 + jnp.exp(-logit))

    pltpu.sync_copy(out_v, out_hbm.at[pl.ds(base, BPW)])


@functools.partial(
    pl.kernel,
    out_type=jax.ShapeDtypeStruct((BATCH,), jnp.float32),
    mesh=plsc.VectorSubcoreMesh(core_axis_name="c", subcore_axis_name="s"),
    compiler_params=pltpu.CompilerParams(
        needs_layout_passes=False, use_tc_tiling_on_sc=True),
    scratch_types=[
        pltpu.VMEM((NLOOK,), jnp.int32),
        pltpu.VMEM((NLOOK,), jnp.int32),
        pltpu.VMEM((NLOOK,), jnp.int32),
        pltpu.VMEM((NLOOK,), jnp.int32),
        pltpu.VMEM((2, BPW, 128), jnp.float32),
        pltpu.VMEM((2, BPW, 128), jnp.float32),
        pltpu.VMEM((BPW * EMB,), jnp.float32),
        pltpu.VMEM((BPW,), jnp.float32),
        pltpu.VMEM((BPW * EMB,), jnp.float32),
        pltpu.VMEM((BPW * EMB,), jnp.float32),
        pltpu.VMEM((BPW,), jnp.float32),
        pltpu.SemaphoreType.DMA,
        pltpu.SemaphoreType.DMA,
    ],
)
def _fm_call(eridx_hbm, esoff_hbm, wridx_hbm, wsoff_hbm, emb_hbm, w_hbm,
             dproj_hbm, dlin_hbm, out_hbm,
             eridx_v, esoff_v, wridx_v, wsoff_v, ebuf_v, wbuf_v,
             dproj_v, dlin_v, accs_v, accq_v, out_v, esem, wsem):
    _fm_body(eridx_hbm, esoff_hbm, wridx_hbm, wsoff_hbm, emb_hbm, w_hbm,
             dproj_hbm, dlin_hbm, out_hbm,
             eridx_v, esoff_v, wridx_v, wsoff_v, ebuf_v, wbuf_v,
             dproj_v, dlin_v, accs_v, accq_v, out_v, esem, wsem)


def kernel(sparse_features, dense_features, sparse_w, sparse_emb,
           dw_W, dw_b, de_W, de_b, bias):
    vt = sparse_features.astype(jnp.int32).reshape(
        NWORK, BPW, NFIELD).transpose(0, 2, 1)
    f_ids = jnp.arange(NFIELD, dtype=jnp.int32)[None, :, None]
    eridx = (f_ids * (VOCAB * EMB // 128) + (vt >> 3)).reshape(NWORK, NLOOK)
    esoff = ((vt & 7) * EMB).reshape(NWORK, NLOOK)
    wridx = (f_ids * (VOCABP // 128) + (vt >> 7)).reshape(NWORK, NLOOK)
    wsoff = (vt & 127).reshape(NWORK, NLOOK)
    emb2d = sparse_emb.reshape(EROWS, 128)
    wpad = jnp.pad(sparse_w.reshape(NFIELD, VOCAB),
                   ((0, 0), (0, VOCABP - VOCAB))).reshape(WROWS, 128)
    dproj = (dense_features @ de_W + de_b).reshape(NWORK, BPW * EMB)
    dlin = (dense_features @ dw_W)[:, 0] + dw_b[0] + bias[0]
    return _fm_call(eridx, esoff, wridx, wsoff, emb2d, wpad, dproj, dlin)


# speedup vs baseline: 2.3161x; 2.3161x over previous
import functools

import jax
import jax.numpy as jnp
from jax import lax
from jax.experimental import pallas as pl
from jax.experimental.pallas import tpu as pltpu
from jax.experimental.pallas import tpu_sc as plsc

NFIELD = 26
VOCAB = 100000
EMB = 16
BATCH = 4096
NCORE = 2
NSUB = 16
NWORK = NCORE * NSUB
BPW = BATCH // NWORK
NLOOK = NFIELD * BPW
FE = BPW * EMB


def _fm_body(vidx_hbm, emb_hbm, w_hbm, dprojt_hbm, dlin_hbm, out_hbm,
             vidx_v, idx_v, ebuf_v, wbuf_v, accs_v, accq_v,
             dprojt_v, dlin_v, out_v, esem, wsem):
    wid = lax.axis_index("s") * NCORE + lax.axis_index("c")
    base = wid * BPW

    pltpu.sync_copy(vidx_hbm.at[wid], vidx_v)
    pltpu.sync_copy(dprojt_hbm.at[wid], dprojt_v)
    pltpu.sync_copy(dlin_hbm.at[pl.ds(base, BPW)], dlin_v)

    for d in range(EMB):
        for g in range(BPW // 16):
            o = d * BPW + g * 16
            row = dprojt_v[pl.ds(o, 16)]
            accs_v[pl.ds(o, 16)] = row
            accq_v[pl.ds(o, 16)] = row * row

    def build_and_fire(f, p):
        fbase = pl.multiple_of(f * BPW, BPW)
        ebase = f * (VOCAB * EMB)
        for d in range(EMB):
            dbase = ebase + d * VOCAB
            for c in range(BPW // 16):
                v16 = vidx_v[pl.ds(fbase + c * 16, 16)]
                idx_v[p, pl.ds(d * BPW + c * 16, 16)] = v16 + dbase
        for d in range(EMB):
            pltpu.async_copy(
                emb_hbm.at[idx_v.at[p, pl.ds(d * BPW, BPW)]],
                ebuf_v.at[p, pl.ds(d * BPW, BPW)], esem)
        wbase = f * VOCAB
        for c in range(BPW // 16):
            v16 = vidx_v[pl.ds(fbase + c * 16, 16)]
            idx_v[p, pl.ds(EMB * BPW + c * 16, 16)] = v16 + wbase
        pltpu.async_copy(
            w_hbm.at[idx_v.at[p, pl.ds(EMB * BPW, BPW)]],
            wbuf_v.at[p], wsem)

    build_and_fire(0, 0)
    build_and_fire(1, 1)

    def field_body(f, carry):
        p = lax.bitwise_and(f, 1)
        for d in range(EMB):
            pltpu.make_async_copy(
                emb_hbm.at[pl.ds(0, BPW)],
                ebuf_v.at[p, pl.ds(d * BPW, BPW)], esem).wait()
        pltpu.make_async_copy(
            w_hbm.at[pl.ds(0, BPW)], wbuf_v.at[p], wsem).wait()

        for d in range(EMB):
            for g in range(BPW // 16):
                o = d * BPW + g * 16
                ev = ebuf_v[p, pl.ds(o, 16)]
                accs_v[pl.ds(o, 16)] = accs_v[pl.ds(o, 16)] + ev
                accq_v[pl.ds(o, 16)] = accq_v[pl.ds(o, 16)] + ev * ev
        for g in range(BPW // 16):
            o = g * 16
            dlin_v[pl.ds(o, 16)] = dlin_v[pl.ds(o, 16)] + wbuf_v[p, pl.ds(o, 16)]

        @pl.when(f + 2 < NFIELD)
        def _():
            build_and_fire(f + 2, p)

        return carry

    lax.fori_loop(0, NFIELD, field_body, 0)

    for g in range(BPW // 16):
        o = g * 16
        fm = jnp.zeros((16,), jnp.float32)
        for d in range(EMB):
            sv = accs_v[pl.ds(d * BPW + o, 16)]
            qv = accq_v[pl.ds(d * BPW + o, 16)]
            fm = fm + (sv * sv - qv)
        logit = dlin_v[pl.ds(o, 16)] + 0.5 * fm
        out_v[pl.ds(o, 16)] = 1.0 / (1.0 + jnp.exp(-logit))

    pltpu.sync_copy(out_v, out_hbm.at[pl.ds(base, BPW)])


@functools.partial(
    pl.kernel,
    out_type=jax.ShapeDtypeStruct((BATCH,), jnp.float32),
    mesh=plsc.VectorSubcoreMesh(core_axis_name="c", subcore_axis_name="s"),
    compiler_params=pltpu.CompilerParams(
        needs_layout_passes=False, use_tc_tiling_on_sc=False),
    scratch_types=[
        pltpu.VMEM((NLOOK,), jnp.int32),
        pltpu.VMEM((2, FE + BPW), jnp.int32),
        pltpu.VMEM((2, FE), jnp.float32),
        pltpu.VMEM((2, BPW), jnp.float32),
        pltpu.VMEM((FE,), jnp.float32),
        pltpu.VMEM((FE,), jnp.float32),
        pltpu.VMEM((FE,), jnp.float32),
        pltpu.VMEM((BPW,), jnp.float32),
        pltpu.VMEM((BPW,), jnp.float32),
        pltpu.SemaphoreType.DMA,
        pltpu.SemaphoreType.DMA,
    ],
)
def _fm_call(vidx_hbm, emb_hbm, w_hbm, dprojt_hbm, dlin_hbm, out_hbm,
             vidx_v, idx_v, ebuf_v, wbuf_v, accs_v, accq_v,
             dprojt_v, dlin_v, out_v, esem, wsem):
    _fm_body(vidx_hbm, emb_hbm, w_hbm, dprojt_hbm, dlin_hbm, out_hbm,
             vidx_v, idx_v, ebuf_v, wbuf_v, accs_v, accq_v,
             dprojt_v, dlin_v, out_v, esem, wsem)


def kernel(sparse_features, dense_features, sparse_w, sparse_emb,
           dw_W, dw_b, de_W, de_b, bias):
    vidx = sparse_features.astype(jnp.int32).reshape(
        NWORK, BPW, NFIELD).transpose(0, 2, 1).reshape(NWORK, NLOOK)
    emb_dm = sparse_emb.transpose(0, 2, 1).reshape(NFIELD * EMB * VOCAB)
    w_dm = sparse_w.transpose(0, 2, 1).reshape(NFIELD * VOCAB)
    dprojt = (dense_features @ de_W + de_b).reshape(
        NWORK, BPW, EMB).transpose(0, 2, 1).reshape(NWORK, FE)
    dlin = (dense_features @ dw_W)[:, 0] + dw_b[0] + bias[0]
    return _fm_call(vidx, emb_dm, w_dm, dprojt, dlin)
